# baseline (speedup 1.0000x reference)
import functools

import jax
import jax.numpy as jnp
from jax import lax
from jax.experimental import pallas as pl
from jax.experimental.pallas import tpu as pltpu
from jax.experimental.pallas import tpu_sc as plsc

_NC = 2
_NS = 16


@functools.partial(jax.jit, static_argnums=(0, 1, 2, 3))
def _sc_input_layer(F, V, D, L, tabt, sidxt, seqt, qidx3, lens):
    B = lens.shape[0]
    NW = _NC * _NS
    BBL = 128
    NBB = B // BBL
    NV = B // 16

    mesh = plsc.VectorSubcoreMesh(core_axis_name="c", subcore_axis_name="s")

    @functools.partial(
        pl.kernel,
        out_type=(
            jax.ShapeDtypeStruct((F * D * (B // 128), 128), jnp.float32),
            jax.ShapeDtypeStruct((D * (B // 128), 128), jnp.float32),
        ),
        mesh=mesh,
        compiler_params=pltpu.CompilerParams(
            use_tc_tiling_on_sc=True, needs_layout_passes=False),
        scratch_types=[
            pltpu.VMEM((V,), jnp.float32),
            pltpu.VMEM((B,), jnp.int32),
            pltpu.VMEM((L, BBL), jnp.int32),
            pltpu.VMEM((B,), jnp.float32),
            pltpu.VMEM((B // 128, 128), jnp.float32),
            pltpu.VMEM((B // 128, 128), jnp.float32),
            pltpu.VMEM((8, 32), jnp.float32),
            pltpu.SemaphoreType.DMA,
            pltpu.SemaphoreType.DMA,
        ],
    )
    def k(tabt_h, sidxt_h, seqt_h, qidx3_h, lens_h, sout_h, pout_h,
          row_v, sidx_v, qidx_v, lens_v, res_v, pres_v, tail_v,
          sem_lo, sem_hi):
        d = lax.axis_index("s") * _NC + lax.axis_index("c")

        pltpu.sync_copy(lens_h, lens_v)
        pltpu.sync_copy(seqt_h.at[d, :], row_v)

        def qblock(bb, carry):
            pltpu.sync_copy(qidx3_h.at[bb], qidx_v)

            def qvec(bv, c2):
                lenv = lens_v[pl.ds(bb * BBL + bv * 16, 16)]
                acc = jnp.zeros((16,), jnp.float32)
                for j in range(L):
                    iv = qidx_v[j, pl.ds(bv * 16, 16)]
                    g = plsc.load_gather(row_v, [iv])
                    m = jnp.minimum(jnp.maximum(lenv - float(j), 0.0), 1.0)
                    acc = acc + g * m
                rv = 1.0 / jnp.maximum(lenv, 1.0)
                b0 = bb * BBL + bv * 16
                pres_v[b0 // 128, pl.ds(b0 % 128, 16)] = acc * rv
                return c2

            lax.fori_loop(0, BBL // 16, qvec, 0)
            return carry

        lax.fori_loop(0, NBB, qblock, 0)
        pltpu.sync_copy(pres_v, pout_h.at[pl.ds(d * (B // 128), B // 128)])

        VH = 50048
        VT = (V // 128) * 128
        VR = VT - VH

        def s_issue(f, h):
            lo, n = (0, VH) if h == 0 else (VH, VR)
            sem = sem_lo if h == 0 else sem_hi
            pltpu.async_copy(tabt_h.at[f, d, pl.ds(lo, n)],
                             row_v.at[pl.ds(lo, n)], sem)

        def s_wait(f, h):
            lo, n = (0, VH) if h == 0 else (VH, VR)
            sem = sem_lo if h == 0 else sem_hi
            pltpu.make_async_copy(tabt_h.at[f, d, pl.ds(lo, n)],
                                  row_v.at[pl.ds(lo, n)], sem).wait()

        def s_tail(f):
            pltpu.sync_copy(
                tabt_h.at[f, pl.ds((d // 8) * 8, 8), pl.ds(VT, V - VT)],
                tail_v)
            dm8 = d % 8
            row_v[pl.ds(VT, 16)] = tail_v[dm8, 0:16]
            row_v[pl.ds(VT + 16, 16)] = tail_v[dm8, 16:32]

        def s_compute(h):
            def svec(bv, c2):
                iv = sidx_v[pl.ds(bv * 16, 16)]
                msel = jnp.minimum(jnp.maximum(iv - (VH - 1), 0), 1)
                if h == 0:
                    loc = jnp.minimum(iv, VH - 1)
                    m = (1 - msel).astype(jnp.float32)
                else:
                    loc = jnp.minimum(jnp.maximum(iv, VH), V - 1)
                    m = msel.astype(jnp.float32)
                g = plsc.load_gather(row_v, [loc]) * m
                if h == 0:
                    res_v[bv // 8, pl.ds((bv % 8) * 16, 16)] = g
                else:
                    prev = res_v[bv // 8, pl.ds((bv % 8) * 16, 16)]
                    res_v[bv // 8, pl.ds((bv % 8) * 16, 16)] = prev + g
                return c2

            lax.fori_loop(0, NV, svec, 0)

        def s_field(f, last):
            s_wait(f, 0)
            s_issue(f, 1)
            pltpu.sync_copy(sidxt_h.at[f], sidx_v)
            s_tail(f)
            s_compute(0)
            s_wait(f, 1)
            if not last:
                s_issue(f + 1, 0)
            s_compute(1)
            fd = f * D + d
            pltpu.sync_copy(res_v, sout_h.at[pl.ds(fd * (B // 128), B // 128)])

        s_issue(0, 0)

        def fbody(f, carry):
            s_field(f, last=False)
            return carry

        lax.fori_loop(0, F - 1, fbody, 0)
        s_field(F - 1, last=True)

    return k(tabt, sidxt, seqt, qidx3, lens)


def kernel(sparse_idx, seq_idx, seq_lengths, dense, sparse_tables, seq_table):
    B, F = sparse_idx.shape
    L = seq_idx.shape[1]
    V, D = seq_table.shape
    tabt = sparse_tables.transpose(0, 2, 1)
    seqt = seq_table.T
    sidxt = sparse_idx.astype(jnp.int32).T
    qidx3 = (seq_idx.astype(jnp.int32).T
             .reshape(L, B // 128, 128).transpose(1, 0, 2))
    lens = seq_lengths.astype(jnp.float32)
    souto, pouto = _sc_input_layer(F, V, D, L, tabt, sidxt, seqt, qidx3, lens)
    sparse_out = souto.reshape(F * D, B).T
    pooled = pouto.reshape(D, B).T
    return jnp.concatenate([sparse_out, pooled, dense], axis=-1)


# speedup vs baseline: 1.1374x; 1.1374x over previous
import functools

import jax
import jax.numpy as jnp
from jax import lax
from jax.experimental import pallas as pl
from jax.experimental.pallas import tpu as pltpu
from jax.experimental.pallas import tpu_sc as plsc

_NC = 2
_NS = 16


@functools.partial(jax.jit, static_argnums=(0, 1, 2, 3))
def _sc_input_layer(F, V, D, L, tabt, sidxt, seqt, qidx3, lens):
    B = lens.shape[0]
    NW = _NC * _NS
    NBB = 16
    BBL = B // NBB
    NV = B // 16

    mesh = plsc.VectorSubcoreMesh(core_axis_name="c", subcore_axis_name="s")

    @functools.partial(
        pl.kernel,
        out_type=(
            jax.ShapeDtypeStruct((F * D * (B // 128), 128), jnp.float32),
            jax.ShapeDtypeStruct((D * (B // 128), 128), jnp.float32),
        ),
        mesh=mesh,
        compiler_params=pltpu.CompilerParams(
            use_tc_tiling_on_sc=True, needs_layout_passes=False),
        scratch_types=[
            pltpu.VMEM((V,), jnp.float32),
            pltpu.VMEM((B,), jnp.int32),
            pltpu.VMEM((L, BBL), jnp.int32),
            pltpu.VMEM((B,), jnp.float32),
            pltpu.VMEM((B // 128, 128), jnp.float32),
            pltpu.VMEM((B // 128, 128), jnp.float32),
        ],
    )
    def k(tabt_h, sidxt_h, seqt_h, qidx3_h, lens_h, sout_h, pout_h,
          row_v, sidx_v, qidx_v, lens_v, res_v, pres_v):
        d = lax.axis_index("s") * _NC + lax.axis_index("c")

        pltpu.sync_copy(lens_h, lens_v)
        pltpu.sync_copy(seqt_h.at[d, :], row_v)

        def qblock(bb, carry):
            pltpu.sync_copy(qidx3_h.at[bb], qidx_v)

            def qvec(bv, c2):
                lenv = lens_v[pl.ds(bb * BBL + bv * 16, 16)]
                acc = jnp.zeros((16,), jnp.float32)
                for j in range(L):
                    iv = qidx_v[j, pl.ds(bv * 16, 16)]
                    g = plsc.load_gather(row_v, [iv])
                    m = jnp.minimum(jnp.maximum(lenv - float(j), 0.0), 1.0)
                    acc = acc + g * m
                rv = 1.0 / jnp.maximum(lenv, 1.0)
                b0 = bb * BBL + bv * 16
                pres_v[b0 // 128, pl.ds(b0 % 128, 16)] = acc * rv
                return c2

            lax.fori_loop(0, BBL // 16, qvec, 0)
            return carry

        lax.fori_loop(0, NBB, qblock, 0)
        pltpu.sync_copy(pres_v, pout_h.at[pl.ds(d * (B // 128), B // 128)])

        def fbody(f, carry):
            pltpu.sync_copy(tabt_h.at[f, d, :], row_v)
            pltpu.sync_copy(sidxt_h.at[f], sidx_v)

            def svec(bv, c2):
                iv = sidx_v[pl.ds(bv * 16, 16)]
                g = plsc.load_gather(row_v, [iv])
                res_v[bv // 8, pl.ds((bv % 8) * 16, 16)] = g
                return c2

            lax.fori_loop(0, NV, svec, 0)
            fd = f * D + d
            pltpu.sync_copy(res_v, sout_h.at[pl.ds(fd * (B // 128), B // 128)])
            return carry

        lax.fori_loop(0, F, fbody, 0)

    return k(tabt, sidxt, seqt, qidx3, lens)


def kernel(sparse_idx, seq_idx, seq_lengths, dense, sparse_tables, seq_table):
    B, F = sparse_idx.shape
    L = seq_idx.shape[1]
    V, D = seq_table.shape
    tabt = sparse_tables.transpose(0, 2, 1)
    seqt = seq_table.T
    sidxt = sparse_idx.astype(jnp.int32).T
    qidx3 = (seq_idx.astype(jnp.int32).T
             .reshape(L, 16, B // 16).transpose(1, 0, 2))
    lens = seq_lengths.astype(jnp.float32)
    souto, pouto = _sc_input_layer(F, V, D, L, tabt, sidxt, seqt, qidx3, lens)
    sparse_out = souto.reshape(F * D, B).T
    pooled = pouto.reshape(D, B).T
    return jnp.concatenate([sparse_out, pooled, dense], axis=-1)


# speedup vs baseline: 1.1737x; 1.0318x over previous
import functools

import jax
import jax.numpy as jnp
from jax import lax
from jax.experimental import pallas as pl
from jax.experimental.pallas import tpu as pltpu
from jax.experimental.pallas import tpu_sc as plsc

_NC = 2
_NS = 16


@functools.partial(jax.jit, static_argnums=(0, 1, 2, 3))
def _sc_input_layer(F, V, D, L, tabt, sidxt, seqt, qidx3, lens):
    B = lens.shape[0]
    NW = _NC * _NS
    BBL = 128
    NBB = B // BBL
    NV = B // 16

    mesh = plsc.VectorSubcoreMesh(core_axis_name="c", subcore_axis_name="s")

    @functools.partial(
        pl.kernel,
        out_type=(
            jax.ShapeDtypeStruct((F * D * (B // 128), 128), jnp.float32),
            jax.ShapeDtypeStruct((D * (B // 128), 128), jnp.float32),
        ),
        mesh=mesh,
        compiler_params=pltpu.CompilerParams(
            use_tc_tiling_on_sc=True, needs_layout_passes=False),
        scratch_types=[
            pltpu.VMEM((V,), jnp.float32),
            pltpu.VMEM((B,), jnp.int32),
            pltpu.VMEM((L, BBL), jnp.int32),
            pltpu.VMEM((L, BBL), jnp.int32),
            pltpu.VMEM((B,), jnp.float32),
            pltpu.VMEM((B // 128, 128), jnp.float32),
            pltpu.VMEM((B // 128, 128), jnp.float32),
            pltpu.SemaphoreType.DMA,
            pltpu.SemaphoreType.DMA,
        ],
    )
    def k(tabt_h, sidxt_h, seqt_h, qidx3_h, lens_h, sout_h, pout_h,
          row_v, sidx_v, qidx_v0, qidx_v1, lens_v, res_v, pres_v,
          qsem0, qsem1):
        d = lax.axis_index("s") * _NC + lax.axis_index("c")

        qbufs = (qidx_v0, qidx_v1)
        qsems = (qsem0, qsem1)

        def q_issue(bb, slot):
            pltpu.async_copy(qidx3_h.at[bb], qbufs[slot], qsems[slot])

        def q_wait(bb, slot):
            pltpu.make_async_copy(
                qidx3_h.at[bb], qbufs[slot], qsems[slot]).wait()

        q_issue(0, 0)
        pltpu.sync_copy(lens_h, lens_v)
        pltpu.sync_copy(seqt_h.at[d, :], row_v)

        def q_compute(bb, buf):
            def qvec(bv, c2):
                lenv = lens_v[pl.ds(bb * BBL + bv * 16, 16)]
                acc = jnp.zeros((16,), jnp.float32)
                for j in range(L):
                    iv = buf[j, pl.ds(bv * 16, 16)]
                    g = plsc.load_gather(row_v, [iv])
                    m = jnp.minimum(jnp.maximum(lenv - float(j), 0.0), 1.0)
                    acc = acc + g * m
                rv = 1.0 / jnp.maximum(lenv, 1.0)
                b0 = bb * BBL + bv * 16
                pres_v[b0 // 128, pl.ds(b0 % 128, 16)] = acc * rv
                return c2

            lax.fori_loop(0, BBL // 16, qvec, 0)

        def qblock(bb2, carry):
            for u in range(2):
                bb = 2 * bb2 + u
                q_wait(bb, u)
                q_issue(bb + 1, 1 - u)
                q_compute(bb, qbufs[u])
            return carry

        lax.fori_loop(0, NBB // 2 - 1, qblock, 0)
        q_wait(NBB - 2, 0)
        q_issue(NBB - 1, 1)
        q_compute(NBB - 2, qbufs[0])
        q_wait(NBB - 1, 1)
        q_compute(NBB - 1, qbufs[1])
        pltpu.sync_copy(pres_v, pout_h.at[pl.ds(d * (B // 128), B // 128)])

        def fbody(f, carry):
            pltpu.sync_copy(tabt_h.at[f, d, :], row_v)
            pltpu.sync_copy(sidxt_h.at[f], sidx_v)

            def svec(bv, c2):
                iv = sidx_v[pl.ds(bv * 16, 16)]
                g = plsc.load_gather(row_v, [iv])
                res_v[bv // 8, pl.ds((bv % 8) * 16, 16)] = g
                return c2

            lax.fori_loop(0, NV, svec, 0)
            fd = f * D + d
            pltpu.sync_copy(res_v, sout_h.at[pl.ds(fd * (B // 128), B // 128)])
            return carry

        lax.fori_loop(0, F, fbody, 0)

    return k(tabt, sidxt, seqt, qidx3, lens)


def kernel(sparse_idx, seq_idx, seq_lengths, dense, sparse_tables, seq_table):
    B, F = sparse_idx.shape
    L = seq_idx.shape[1]
    V, D = seq_table.shape
    tabt = sparse_tables.transpose(0, 2, 1)
    seqt = seq_table.T
    sidxt = sparse_idx.astype(jnp.int32).T
    qidx3 = (seq_idx.astype(jnp.int32).T
             .reshape(L, B // 128, 128).transpose(1, 0, 2))
    lens = seq_lengths.astype(jnp.float32)
    souto, pouto = _sc_input_layer(F, V, D, L, tabt, sidxt, seqt, qidx3, lens)
    sparse_out = souto.reshape(F * D, B).T
    pooled = pouto.reshape(D, B).T
    return jnp.concatenate([sparse_out, pooled, dense], axis=-1)


# speedup vs baseline: 1.2955x; 1.1038x over previous
import functools

import jax
import jax.numpy as jnp
from jax import lax
from jax.experimental import pallas as pl
from jax.experimental.pallas import tpu as pltpu
from jax.experimental.pallas import tpu_sc as plsc

_NC = 2
_NS = 16


@functools.partial(jax.jit, static_argnums=(0, 1, 2, 3))
def _sc_input_layer(F, V, D, L, tabt, sidxt, seqt, qidx3, lens):
    B = lens.shape[0]
    NW = _NC * _NS
    BBL = 128
    NBB = B // BBL
    NV = B // 16

    mesh = plsc.VectorSubcoreMesh(core_axis_name="c", subcore_axis_name="s")

    @functools.partial(
        pl.kernel,
        out_type=(
            jax.ShapeDtypeStruct((F * D * (B // 128), 128), jnp.float32),
            jax.ShapeDtypeStruct((D * (B // 128), 128), jnp.float32),
        ),
        mesh=mesh,
        compiler_params=pltpu.CompilerParams(
            use_tc_tiling_on_sc=True, needs_layout_passes=False),
        scratch_types=[
            pltpu.VMEM((V,), jnp.float32),
            pltpu.VMEM((B,), jnp.int32),
            pltpu.VMEM((L, BBL), jnp.int32),
            pltpu.VMEM((L, BBL), jnp.int32),
            pltpu.VMEM((B,), jnp.float32),
            pltpu.VMEM((B // 128, 128), jnp.float32),
            pltpu.VMEM((B // 128, 128), jnp.float32),
            pltpu.SemaphoreType.DMA,
            pltpu.SemaphoreType.DMA,
        ],
    )
    def k(tabt_h, sidxt_h, seqt_h, qidx3_h, lens_h, sout_h, pout_h,
          row_v, sidx_v, qidx_v0, qidx_v1, lens_v, res_v, pres_v,
          qsem0, qsem1):
        d = lax.axis_index("s") * _NC + lax.axis_index("c")

        qbufs = (qidx_v0, qidx_v1)
        qsems = (qsem0, qsem1)

        def q_issue(bb, slot):
            pltpu.async_copy(qidx3_h.at[bb], qbufs[slot], qsems[slot])

        def q_wait(bb, slot):
            pltpu.make_async_copy(
                qidx3_h.at[bb], qbufs[slot], qsems[slot]).wait()

        q_issue(0, 0)
        pltpu.sync_copy(lens_h, lens_v)
        pltpu.sync_copy(seqt_h.at[d, :], row_v)

        def q_compute(bb, buf):
            def qvec(bv, c2):
                lenv = lens_v[pl.ds(bb * BBL + bv * 16, 16)]
                acc = jnp.zeros((16,), jnp.float32)
                for j in range(L):
                    iv = buf[j, pl.ds(bv * 16, 16)]
                    g = plsc.load_gather(row_v, [iv])
                    m = jnp.minimum(jnp.maximum(lenv - float(j), 0.0), 1.0)
                    acc = acc + g * m
                rv = 1.0 / jnp.maximum(lenv, 1.0)
                b0 = bb * BBL + bv * 16
                pres_v[b0 // 128, pl.ds(b0 % 128, 16)] = acc * rv
                return c2

            lax.fori_loop(0, BBL // 16, qvec, 0)

        def qblock(bb2, carry):
            for u in range(2):
                bb = 2 * bb2 + u
                q_wait(bb, u)
                q_issue(bb + 1, 1 - u)
                q_compute(bb, qbufs[u])
            return carry

        lax.fori_loop(0, NBB // 2 - 1, qblock, 0)
        q_wait(NBB - 2, 0)
        q_issue(NBB - 1, 1)
        q_compute(NBB - 2, qbufs[0])
        q_wait(NBB - 1, 1)
        q_compute(NBB - 1, qbufs[1])
        pltpu.sync_copy(pres_v, pout_h.at[pl.ds(d * (B // 128), B // 128)])

        def fbody(f, carry):
            pltpu.sync_copy(tabt_h.at[f, d, :], row_v)
            pltpu.sync_copy(sidxt_h.at[f], sidx_v)

            def svec(bv8, c2):
                for u in range(8):
                    bv = bv8 * 8 + u
                    iv = sidx_v[pl.ds(bv * 16, 16)]
                    g = plsc.load_gather(row_v, [iv])
                    res_v[bv8, pl.ds(u * 16, 16)] = g
                return c2

            lax.fori_loop(0, NV // 8, svec, 0)
            fd = f * D + d
            pltpu.sync_copy(res_v, sout_h.at[pl.ds(fd * (B // 128), B // 128)])
            return carry

        lax.fori_loop(0, F, fbody, 0)

    return k(tabt, sidxt, seqt, qidx3, lens)


def kernel(sparse_idx, seq_idx, seq_lengths, dense, sparse_tables, seq_table):
    B, F = sparse_idx.shape
    L = seq_idx.shape[1]
    V, D = seq_table.shape
    tabt = sparse_tables.transpose(0, 2, 1)
    seqt = seq_table.T
    sidxt = sparse_idx.astype(jnp.int32).T
    qidx3 = (seq_idx.astype(jnp.int32).T
             .reshape(L, B // 128, 128).transpose(1, 0, 2))
    lens = seq_lengths.astype(jnp.float32)
    souto, pouto = _sc_input_layer(F, V, D, L, tabt, sidxt, seqt, qidx3, lens)
    sparse_out = souto.reshape(F * D, B).T
    pooled = pouto.reshape(D, B).T
    return jnp.concatenate([sparse_out, pooled, dense], axis=-1)


# speedup vs baseline: 1.2967x; 1.0009x over previous
import functools

import jax
import jax.numpy as jnp
from jax import lax
from jax.experimental import pallas as pl
from jax.experimental.pallas import tpu as pltpu
from jax.experimental.pallas import tpu_sc as plsc

_NC = 2
_NS = 16


@functools.partial(jax.jit, static_argnums=(0, 1, 2, 3))
def _sc_input_layer(F, V, D, L, tabt, sidxt, seqt, qidx3, lens):
    B = lens.shape[0]
    NW = _NC * _NS
    BBL = 128
    NBB = B // BBL
    NV = B // 16

    mesh = plsc.VectorSubcoreMesh(core_axis_name="c", subcore_axis_name="s")

    @functools.partial(
        pl.kernel,
        out_type=(
            jax.ShapeDtypeStruct((F * D * (B // 128), 128), jnp.float32),
            jax.ShapeDtypeStruct((D * (B // 128), 128), jnp.float32),
        ),
        mesh=mesh,
        compiler_params=pltpu.CompilerParams(
            use_tc_tiling_on_sc=True, needs_layout_passes=False),
        scratch_types=[
            pltpu.VMEM((V,), jnp.float32),
            pltpu.VMEM((B,), jnp.int32),
            pltpu.VMEM((L, BBL), jnp.int32),
            pltpu.VMEM((L, BBL), jnp.int32),
            pltpu.VMEM((B,), jnp.float32),
            pltpu.VMEM((B // 128, 128), jnp.float32),
            pltpu.VMEM((B // 128, 128), jnp.float32),
            pltpu.SemaphoreType.DMA,
            pltpu.SemaphoreType.DMA,
        ],
    )
    def k(tabt_h, sidxt_h, seqt_h, qidx3_h, lens_h, sout_h, pout_h,
          row_v, sidx_v, qidx_v0, qidx_v1, lens_v, res_v, pres_v,
          qsem0, qsem1):
        d = lax.axis_index("s") * _NC + lax.axis_index("c")

        qbufs = (qidx_v0, qidx_v1)
        qsems = (qsem0, qsem1)

        def q_issue(bb, slot):
            pltpu.async_copy(qidx3_h.at[bb], qbufs[slot], qsems[slot])

        def q_wait(bb, slot):
            pltpu.make_async_copy(
                qidx3_h.at[bb], qbufs[slot], qsems[slot]).wait()

        q_issue(0, 0)
        pltpu.sync_copy(lens_h, lens_v)
        pltpu.sync_copy(seqt_h.at[d, :], row_v)

        def q_compute(bb, buf):
            def qvec(bv, c2):
                lenv = lens_v[pl.ds(bb * BBL + bv * 16, 16)]
                acc = jnp.zeros((16,), jnp.float32)
                for j in range(L):
                    iv = buf[j, pl.ds(bv * 16, 16)]
                    g = plsc.load_gather(row_v, [iv])
                    m = jnp.minimum(jnp.maximum(lenv - float(j), 0.0), 1.0)
                    acc = acc + g * m
                rv = 1.0 / jnp.maximum(lenv, 1.0)
                b0 = bb * BBL + bv * 16
                pres_v[b0 // 128, pl.ds(b0 % 128, 16)] = acc * rv
                return c2

            lax.fori_loop(0, BBL // 16, qvec, 0)

        def qblock(bb2, carry):
            for u in range(2):
                bb = 2 * bb2 + u
                q_wait(bb, u)
                q_issue(bb + 1, 1 - u)
                q_compute(bb, qbufs[u])
            return carry

        lax.fori_loop(0, NBB // 2 - 1, qblock, 0)
        q_wait(NBB - 2, 0)
        q_issue(NBB - 1, 1)
        q_compute(NBB - 2, qbufs[0])
        q_wait(NBB - 1, 1)
        q_compute(NBB - 1, qbufs[1])
        pltpu.sync_copy(pres_v, pout_h.at[pl.ds(d * (B // 128), B // 128)])

        def fbody(f, carry):
            pltpu.sync_copy(tabt_h.at[f, d, :], row_v)
            pltpu.sync_copy(sidxt_h.at[f], sidx_v)

            def svec(bq, c2):
                for r in range(2):
                    for u in range(8):
                        bv = bq * 16 + r * 8 + u
                        iv = sidx_v[pl.ds(bv * 16, 16)]
                        g = plsc.load_gather(row_v, [iv])
                        res_v[bq * 2 + r, pl.ds(u * 16, 16)] = g
                return c2

            lax.fori_loop(0, NV // 16, svec, 0)
            fd = f * D + d
            pltpu.sync_copy(res_v, sout_h.at[pl.ds(fd * (B // 128), B // 128)])
            return carry

        lax.fori_loop(0, F, fbody, 0)

    return k(tabt, sidxt, seqt, qidx3, lens)


def kernel(sparse_idx, seq_idx, seq_lengths, dense, sparse_tables, seq_table):
    B, F = sparse_idx.shape
    L = seq_idx.shape[1]
    V, D = seq_table.shape
    tabt = sparse_tables.transpose(0, 2, 1)
    seqt = seq_table.T
    sidxt = sparse_idx.astype(jnp.int32).T
    qidx3 = (seq_idx.astype(jnp.int32).T
             .reshape(L, B // 128, 128).transpose(1, 0, 2))
    lens = seq_lengths.astype(jnp.float32)
    souto, pouto = _sc_input_layer(F, V, D, L, tabt, sidxt, seqt, qidx3, lens)
    sparse_out = souto.reshape(F * D, B).T
    pooled = pouto.reshape(D, B).T
    return jnp.concatenate([sparse_out, pooled, dense], axis=-1)
